# baseline (speedup 1.0000x reference)
import jax
import jax.numpy as jnp
from jax.experimental import pallas as pl

B = 256
N = 16384
NUM_WRITES = 4
NUM_READS = 8

B_BLK = 16
N_BLK = 1024


def _body(ww_ref, fg_ref, rw_ref, pu_ref, out_ref):
    pu = pu_ref[...]
    p = (1.0 - ww_ref[:, 0, :]) * (1.0 - ww_ref[:, 1, :])
    p = p * (1.0 - ww_ref[:, 2, :]) * (1.0 - ww_ref[:, 3, :])
    usage = 1.0 - (1.0 - pu) * p
    fg = fg_ref[...]
    phi = usage
    for r in range(NUM_READS):
        phi = phi * (1.0 - fg[:, r:r + 1] * rw_ref[:, r, :])
    out_ref[...] = phi


def kernel(write_weights, free_gate, read_weights, prev_usage):
    grid = (B // B_BLK, N // N_BLK)
    return pl.pallas_call(
        _body,
        grid=grid,
        in_specs=[
            pl.BlockSpec((B_BLK, NUM_WRITES, N_BLK), lambda i, j: (i, 0, j)),
            pl.BlockSpec((B_BLK, NUM_READS), lambda i, j: (i, 0)),
            pl.BlockSpec((B_BLK, NUM_READS, N_BLK), lambda i, j: (i, 0, j)),
            pl.BlockSpec((B_BLK, N_BLK), lambda i, j: (i, j)),
        ],
        out_specs=pl.BlockSpec((B_BLK, N_BLK), lambda i, j: (i, j)),
        out_shape=jax.ShapeDtypeStruct((B, N), jnp.float32),
    )(write_weights, free_gate, read_weights, prev_usage)


# speedup vs baseline: 1.6624x; 1.6624x over previous
import jax
import jax.numpy as jnp
from jax.experimental import pallas as pl

B = 256
N = 16384
NUM_WRITES = 4
NUM_READS = 8

B_BLK = 32
N_BLK = 2048


def _body(ww_ref, fg_ref, rw_ref, pu_ref, out_ref):
    pu = pu_ref[...]
    p = (1.0 - ww_ref[:, 0, :]) * (1.0 - ww_ref[:, 1, :])
    p = p * (1.0 - ww_ref[:, 2, :]) * (1.0 - ww_ref[:, 3, :])
    usage = 1.0 - (1.0 - pu) * p
    fg = fg_ref[...]
    phi = usage
    for r in range(NUM_READS):
        phi = phi * (1.0 - fg[:, r:r + 1] * rw_ref[:, r, :])
    out_ref[...] = phi


def kernel(write_weights, free_gate, read_weights, prev_usage):
    grid = (B // B_BLK, N // N_BLK)
    return pl.pallas_call(
        _body,
        grid=grid,
        in_specs=[
            pl.BlockSpec((B_BLK, NUM_WRITES, N_BLK), lambda i, j: (i, 0, j)),
            pl.BlockSpec((B_BLK, NUM_READS), lambda i, j: (i, 0)),
            pl.BlockSpec((B_BLK, NUM_READS, N_BLK), lambda i, j: (i, 0, j)),
            pl.BlockSpec((B_BLK, N_BLK), lambda i, j: (i, j)),
        ],
        out_specs=pl.BlockSpec((B_BLK, N_BLK), lambda i, j: (i, j)),
        out_shape=jax.ShapeDtypeStruct((B, N), jnp.float32),
    )(write_weights, free_gate, read_weights, prev_usage)
